# Initial kernel scaffold; baseline (speedup 1.0000x reference)
import jax
import jax.numpy as jnp
from jax.experimental import pallas as pl


def kernel(x, edge_index, Wl1, bl1, Wr1, Wl2, bl2, Wr2):
    raise NotImplementedError("write your pallas kernel here")


# speedup vs baseline: 5.0202x; 5.0202x over previous
import jax
import jax.numpy as jnp
from jax import lax
from jax.experimental import pallas as pl
from jax.experimental.pallas import tpu as pltpu
from jax.experimental.pallas import tpu_sc as plsc

N = 10000
NP = 10240
E = 320000
K = 11
NC, NS = 2, 16
NW = NC * NS
EPW = E // NW
C = 80
NCHUNK = EPW // C
RPT = NP // NS


def _make_segsum(D, with_deg):
    ZR = 128
    mesh = plsc.VectorSubcoreMesh(core_axis_name="c", subcore_axis_name="s")
    out_type = [jax.ShapeDtypeStruct((NC, NP, D), jnp.float32)]
    scratch = [
        pltpu.VMEM((C,), jnp.int32),
        pltpu.VMEM((C,), jnp.int32),
        pltpu.VMEM((C, D), jnp.float32),
        pltpu.VMEM((ZR, D), jnp.float32),
        pltpu.VMEM_SHARED((NP, D), jnp.float32),
        pltpu.SemaphoreType.DMA,
    ]
    if with_deg:
        out_type.append(jax.ShapeDtypeStruct((NC, NP), jnp.float32))
        scratch += [
            pltpu.VMEM((C,), jnp.float32),
            pltpu.VMEM((RPT,), jnp.float32),
            pltpu.VMEM_SHARED((NP,), jnp.float32),
        ]

    def body(x_hbm, src_hbm, dst_hbm, *rest):
        if with_deg:
            (h_out, deg_out, src_v, dst_v, rows_v, z_v, h_sh, sem,
             ones_v, dz_v, deg_sh) = rest
        else:
            h_out, src_v, dst_v, rows_v, z_v, h_sh, sem = rest
        c = lax.axis_index("c")
        s = lax.axis_index("s")
        wid = c * NS + s
        row0 = s * RPT

        zero16 = jnp.zeros((16,), jnp.float32)

        @pl.loop(0, ZR)
        def _zero_zv(r):
            for j in range(D // 16):
                z_v[r, pl.ds(j * 16, 16)] = zero16

        @pl.loop(0, RPT // ZR)
        def _zero_h(k):
            pltpu.sync_copy(z_v, h_sh.at[pl.ds(row0 + k * ZR, ZR)])

        if with_deg:
            @pl.loop(0, RPT // 16)
            def _zero_dz(i):
                dz_v[pl.ds(i * 16, 16)] = zero16

            for j in range(C // 16):
                ones_v[pl.ds(j * 16, 16)] = jnp.ones((16,), jnp.float32)
            pltpu.sync_copy(dz_v, deg_sh.at[pl.ds(row0, RPT)])

        plsc.subcore_barrier()

        base = wid * EPW

        @pl.loop(0, NCHUNK)
        def _edges(i):
            off = base + i * C
            pltpu.sync_copy(src_hbm.at[pl.ds(off, C)], src_v)
            pltpu.sync_copy(dst_hbm.at[pl.ds(off, C)], dst_v)
            pltpu.async_copy(x_hbm.at[src_v], rows_v, sem).wait()
            pltpu.sync_copy(rows_v, h_sh.at[dst_v], add=True)
            if with_deg:
                pltpu.sync_copy(ones_v, deg_sh.at[dst_v], add=True)

        plsc.subcore_barrier()

        pltpu.sync_copy(h_sh.at[pl.ds(row0, RPT)], h_out.at[c, pl.ds(row0, RPT)])
        if with_deg:
            pltpu.sync_copy(deg_sh.at[pl.ds(row0, RPT)],
                            deg_out.at[c, pl.ds(row0, RPT)])

    return pl.kernel(body, out_type=tuple(out_type), mesh=mesh,
                     scratch_types=tuple(scratch),
                     compiler_params=pltpu.CompilerParams(
                         use_tc_tiling_on_sc=False))


_segsum_cache = {}


def _segsum(D, with_deg):
    key = (D, with_deg)
    if key not in _segsum_cache:
        _segsum_cache[key] = _make_segsum(D, with_deg)
    return _segsum_cache[key]


def _mfconv_tc(hp, xin, wlf, blf, wrf, degcol, hout, relu):
    B = 1024
    din = xin.shape[1]
    kh = wlf.shape[1]
    grid = (pl.cdiv(N, B),)

    def tc_body(h0, h1, xr, wl, bl, wr, d0, d1, out):
        h = h0[0] + h1[0]
        r = jnp.dot(h, wl[...], preferred_element_type=jnp.float32)
        r += jnp.dot(xr[...], wr[...], preferred_element_type=jnp.float32)
        r += bl[...]
        deg = jnp.minimum(d0[0] + d1[0], float(K - 1))
        grp = (lax.broadcasted_iota(jnp.int32, (1, kh), 1) // hout
               ).astype(jnp.float32)
        m = r * (deg == grp).astype(jnp.float32)
        if hout % 128 == 0:
            acc = m[:, 0:hout]
            for d in range(1, K):
                acc += m[:, d * hout:(d + 1) * hout]
        else:
            row = lax.broadcasted_iota(jnp.int32, (kh, hout), 0)
            col = lax.broadcasted_iota(jnp.int32, (kh, hout), 1)
            g = (row % hout == col).astype(jnp.float32)
            acc = jnp.dot(m, g, preferred_element_type=jnp.float32)
        out[...] = jnp.maximum(acc, 0.0) if relu else acc

    return pl.pallas_call(
        tc_body,
        grid=grid,
        in_specs=[
            pl.BlockSpec((1, B, hp.shape[2]), lambda i: (0, i, 0)),
            pl.BlockSpec((1, B, hp.shape[2]), lambda i: (1, i, 0)),
            pl.BlockSpec((B, din), lambda i: (i, 0)),
            pl.BlockSpec((din, kh), lambda i: (0, 0)),
            pl.BlockSpec((1, kh), lambda i: (0, 0)),
            pl.BlockSpec((din, kh), lambda i: (0, 0)),
            pl.BlockSpec((1, B, 1), lambda i: (0, i, 0)),
            pl.BlockSpec((1, B, 1), lambda i: (1, i, 0)),
        ],
        out_specs=pl.BlockSpec((B, hout), lambda i: (i, 0)),
        out_shape=jax.ShapeDtypeStruct((N, hout), jnp.float32),
    )(hp, hp, xin, wlf, blf, wrf, degcol, degcol)


def kernel(x, edge_index, Wl1, bl1, Wr1, Wl2, bl2, Wr2):
    ei = edge_index.astype(jnp.int32)
    src, dst = ei[0], ei[1]

    wl1f = Wl1.transpose(1, 0, 2).reshape(128, K * 32)
    wr1f = Wr1.transpose(1, 0, 2).reshape(128, K * 32)
    bl1f = bl1.reshape(1, K * 32)
    wl2f = Wl2.transpose(1, 0, 2).reshape(32, K * 128)
    wr2f = Wr2.transpose(1, 0, 2).reshape(32, K * 128)
    bl2f = bl2.reshape(1, K * 128)

    hp1, degp = _segsum(128, True)(x, src, dst)
    degcol = degp.reshape(NC, NP, 1)
    o1 = _mfconv_tc(hp1, x, wl1f, bl1f, wr1f, degcol, 32, relu=True)
    (hp2,) = _segsum(32, False)(o1, src, dst)
    out = _mfconv_tc(hp2, o1, wl2f, bl2f, wr2f, degcol, 128, relu=False)
    return out


# speedup vs baseline: 11.4686x; 2.2845x over previous
import jax
import jax.numpy as jnp
from jax import lax
from jax.experimental import pallas as pl
from jax.experimental.pallas import tpu as pltpu
from jax.experimental.pallas import tpu_sc as plsc

N = 10000
NP = 10240
E = 320000
K = 11
NC, NS = 2, 16
NW = NC * NS
EPW = E // NW
C = 80
NCHUNK = EPW // C
RPT = NP // NS


NPAIR = (NCHUNK - 1) // 2


def _make_segsum(D, with_deg):
    mesh = plsc.VectorSubcoreMesh(core_axis_name="c", subcore_axis_name="s")
    out_type = [jax.ShapeDtypeStruct((NC, NP, D), jnp.float32)]
    scratch = [
        pltpu.VMEM((NCHUNK, C), jnp.int32),
        pltpu.VMEM((NCHUNK, C), jnp.int32),
        pltpu.VMEM((C, D), jnp.float32),
        pltpu.VMEM((C, D), jnp.float32),
        pltpu.VMEM_SHARED((NP, D), jnp.float32),
        pltpu.SemaphoreType.DMA,
        pltpu.SemaphoreType.DMA,
        pltpu.SemaphoreType.DMA,
        pltpu.SemaphoreType.DMA,
    ]
    if with_deg:
        out_type.append(jax.ShapeDtypeStruct((NC, NP), jnp.float32))
        scratch += [
            pltpu.VMEM((C,), jnp.float32),
            pltpu.VMEM((RPT,), jnp.float32),
            pltpu.VMEM_SHARED((NP,), jnp.float32),
            pltpu.SemaphoreType.DMA,
        ]

    def body(x_hbm, src_hbm, dst_hbm, *rest):
        if with_deg:
            (h_out, deg_out, src_all, dst_all, buf0, buf1, h_sh,
             gs0, gs1, ss0, ss1, ones_v, dz_v, deg_sh, ds0) = rest
        else:
            (h_out, src_all, dst_all, buf0, buf1, h_sh,
             gs0, gs1, ss0, ss1) = rest
        c = lax.axis_index("c")
        s = lax.axis_index("s")
        wid = c * NS + s
        row0 = s * RPT

        zero16 = jnp.zeros((16,), jnp.float32)

        @pl.loop(0, C)
        def _zero_buf0(r):
            for j in range(D // 16):
                buf0[r, pl.ds(j * 16, 16)] = zero16

        pltpu.sync_copy(src_hbm.at[wid], src_all)
        pltpu.sync_copy(dst_hbm.at[wid], dst_all)

        @pl.loop(0, RPT // C)
        def _zero_h(k):
            pltpu.sync_copy(buf0, h_sh.at[pl.ds(row0 + k * C, C)])

        if with_deg:
            @pl.loop(0, RPT // 16)
            def _zero_dz(i):
                dz_v[pl.ds(i * 16, 16)] = zero16

            for j in range(C // 16):
                ones_v[pl.ds(j * 16, 16)] = jnp.ones((16,), jnp.float32)
            pltpu.sync_copy(dz_v, deg_sh.at[pl.ds(row0, RPT)])

        plsc.subcore_barrier()

        def g_issue(i, buf, sem):
            pltpu.async_copy(x_hbm.at[src_all.at[i]], buf, sem)

        def g_wait(i, buf, sem):
            pltpu.make_async_copy(x_hbm.at[src_all.at[i]], buf, sem).wait()

        def s_issue(i, buf, sem):
            pltpu.async_copy(buf, h_sh.at[dst_all.at[i]], sem, add=True)

        def s_wait(i, buf, sem):
            pltpu.make_async_copy(buf, h_sh.at[dst_all.at[i]], sem).wait()

        def d_issue(i):
            pltpu.async_copy(ones_v, deg_sh.at[dst_all.at[i]], ds0, add=True)

        def d_wait(i):
            pltpu.make_async_copy(ones_v, deg_sh.at[dst_all.at[i]], ds0).wait()

        g_issue(0, buf0, gs0)
        g_issue(1, buf1, gs1)

        @pl.loop(0, NPAIR)
        def _pairs(k):
            e = k * 2
            g_wait(e, buf0, gs0)
            s_issue(e, buf0, ss0)
            if with_deg:
                d_issue(e)
            s_wait(e, buf0, ss0)
            g_issue(e + 2, buf0, gs0)
            g_wait(e + 1, buf1, gs1)
            s_issue(e + 1, buf1, ss1)
            if with_deg:
                d_issue(e + 1)
            s_wait(e + 1, buf1, ss1)

            @pl.when(k < NPAIR - 1)
            def _prefetch_odd():
                g_issue(e + 3, buf1, gs1)

            if with_deg:
                d_wait(e)
                d_wait(e + 1)

        last = NCHUNK - 1
        g_wait(last, buf0, gs0)
        s_issue(last, buf0, ss0)
        if with_deg:
            d_issue(last)
        s_wait(last, buf0, ss0)
        if with_deg:
            d_wait(last)

        plsc.subcore_barrier()

        pltpu.sync_copy(h_sh.at[pl.ds(row0, RPT)], h_out.at[c, pl.ds(row0, RPT)])
        if with_deg:
            pltpu.sync_copy(deg_sh.at[pl.ds(row0, RPT)],
                            deg_out.at[c, pl.ds(row0, RPT)])

    return pl.kernel(body, out_type=tuple(out_type), mesh=mesh,
                     scratch_types=tuple(scratch),
                     compiler_params=pltpu.CompilerParams(
                         use_tc_tiling_on_sc=False))


_segsum_cache = {}


def _segsum(D, with_deg):
    key = (D, with_deg)
    if key not in _segsum_cache:
        _segsum_cache[key] = _make_segsum(D, with_deg)
    return _segsum_cache[key]


def _mfconv_tc(hp, xin, wlf, blf, wrf, degcol, hout, relu):
    B = 1024
    din = xin.shape[1]
    kh = wlf.shape[1]
    grid = (pl.cdiv(N, B),)

    def tc_body(h0, h1, xr, wl, bl, wr, d0, d1, out):
        h = h0[0] + h1[0]
        r = jnp.dot(h, wl[...], preferred_element_type=jnp.float32)
        r += jnp.dot(xr[...], wr[...], preferred_element_type=jnp.float32)
        r += bl[...]
        deg = jnp.minimum(d0[0] + d1[0], float(K - 1))
        grp = (lax.broadcasted_iota(jnp.int32, (1, kh), 1) // hout
               ).astype(jnp.float32)
        m = r * (deg == grp).astype(jnp.float32)
        if hout % 128 == 0:
            acc = m[:, 0:hout]
            for d in range(1, K):
                acc += m[:, d * hout:(d + 1) * hout]
        else:
            row = lax.broadcasted_iota(jnp.int32, (kh, hout), 0)
            col = lax.broadcasted_iota(jnp.int32, (kh, hout), 1)
            g = (row % hout == col).astype(jnp.float32)
            acc = jnp.dot(m, g, preferred_element_type=jnp.float32)
        out[...] = jnp.maximum(acc, 0.0) if relu else acc

    return pl.pallas_call(
        tc_body,
        grid=grid,
        in_specs=[
            pl.BlockSpec((1, B, hp.shape[2]), lambda i: (0, i, 0)),
            pl.BlockSpec((1, B, hp.shape[2]), lambda i: (1, i, 0)),
            pl.BlockSpec((B, din), lambda i: (i, 0)),
            pl.BlockSpec((din, kh), lambda i: (0, 0)),
            pl.BlockSpec((1, kh), lambda i: (0, 0)),
            pl.BlockSpec((din, kh), lambda i: (0, 0)),
            pl.BlockSpec((1, B, 1), lambda i: (0, i, 0)),
            pl.BlockSpec((1, B, 1), lambda i: (1, i, 0)),
        ],
        out_specs=pl.BlockSpec((B, hout), lambda i: (i, 0)),
        out_shape=jax.ShapeDtypeStruct((N, hout), jnp.float32),
    )(hp, hp, xin, wlf, blf, wrf, degcol, degcol)


def kernel(x, edge_index, Wl1, bl1, Wr1, Wl2, bl2, Wr2):
    ei = edge_index.astype(jnp.int32)
    src = ei[0].reshape(NW, NCHUNK, C)
    dst = ei[1].reshape(NW, NCHUNK, C)

    wl1f = Wl1.transpose(1, 0, 2).reshape(128, K * 32)
    wr1f = Wr1.transpose(1, 0, 2).reshape(128, K * 32)
    bl1f = bl1.reshape(1, K * 32)
    wl2f = Wl2.transpose(1, 0, 2).reshape(32, K * 128)
    wr2f = Wr2.transpose(1, 0, 2).reshape(32, K * 128)
    bl2f = bl2.reshape(1, K * 128)

    hp1, degp = _segsum(128, True)(x, src, dst)
    degcol = degp.reshape(NC, NP, 1)
    o1 = _mfconv_tc(hp1, x, wl1f, bl1f, wr1f, degcol, 32, relu=True)
    (hp2,) = _segsum(32, False)(o1, src, dst)
    out = _mfconv_tc(hp2, o1, wl2f, bl2f, wr2f, degcol, 128, relu=False)
    return out
